# baseline (speedup 1.0000x reference)
import jax
import jax.numpy as jnp
from jax import lax
from jax.experimental import pallas as pl
from jax.experimental.pallas import tpu as pltpu
from jax.experimental.pallas import tpu_sc as plsc

N = 10000
E = 320000
D = 128
HD = D // 2

NC = 2
NS = 16
CH = 128
NCHUNK = 160
EPW = NCHUNK * CH
EPAD = NS * EPW
ACC_N = 10240
RPT = ACC_N // NS
RSTAGE = 128
NBUF = 4
HCHUNK = 80


def _sc_aggregate_body(xs_hbm, srcs_hbm, dsts_hbm, ws_hbm, out_hbm,
                       src_v, dst_v, w_v, buf0, buf1, buf2, buf3,
                       stage, acc, gsem, ssem):
    bufs = [buf0, buf1, buf2, buf3]
    cid = lax.axis_index("c")
    sid = lax.axis_index("s")

    zero16 = jnp.zeros((16,), jnp.float32)

    def _zero_row(i, _):
        for r in range(HD // 16):
            stage[i, pl.ds(r * 16, 16)] = zero16
        return 0

    lax.fori_loop(0, RSTAGE, _zero_row, 0)
    for p in range(RPT // RSTAGE):
        pltpu.sync_copy(stage, acc.at[pl.ds(sid * RPT + p * RSTAGE, RSTAGE)])
    plsc.subcore_barrier()

    def _gather(c, rows):
        pltpu.async_copy(xs_hbm.at[src_v.at[c]], rows, gsem)

    def _wait_gather(rows):
        pltpu.make_async_copy(xs_hbm.at[src_v.at[0]], rows, gsem).wait()

    def _scatter(c, rows):
        pltpu.async_copy(rows, acc.at[dst_v.at[c]], ssem, add=True)

    def _wait_scatter(rows):
        pltpu.make_async_copy(rows, acc.at[dst_v.at[0]], ssem).wait()

    def _scale(c, rows):
        def _edge_group(g, _):
            wv = w_v[c, pl.ds(g * 16, 16)]
            for l in range(16):
                j = g * 16 + l
                w = wv[l]
                for r in range(HD // 16):
                    rows[j, pl.ds(r * 16, 16)] = rows[j, pl.ds(r * 16, 16)] * w
            return 0

        lax.fori_loop(0, CH // 16, _edge_group, 0)

    for h in range(NCHUNK // HCHUNK):
        pltpu.sync_copy(srcs_hbm.at[cid, sid, pl.ds(h * HCHUNK, HCHUNK)],
                        src_v)
        pltpu.sync_copy(dsts_hbm.at[sid, pl.ds(h * HCHUNK, HCHUNK)], dst_v)
        pltpu.sync_copy(ws_hbm.at[sid, pl.ds(h * HCHUNK, HCHUNK)], w_v)

        for b in range(NBUF - 1):
            _gather(b, bufs[b])

        _wait_gather(bufs[0])
        _scale(0, bufs[0])
        _scatter(0, bufs[0])
        _gather(NBUF - 1, bufs[NBUF - 1])

        def _step(c, i):
            _wait_gather(bufs[i])
            _scale(c, bufs[i])
            _scatter(c, bufs[i])
            _wait_scatter(bufs[(i + 3) % NBUF])
            _gather(c + NBUF - 1, bufs[(i + 3) % NBUF])

        def _quad(t, _):
            for i in range(NBUF):
                c = t * NBUF + 1 + i
                _step(c, (1 + i) % NBUF)
            return 0

        lax.fori_loop(0, (HCHUNK - NBUF) // NBUF, _quad, 0)

        for c in range(HCHUNK - 3, HCHUNK):
            i = c % NBUF
            _wait_gather(bufs[i])
            _scale(c, bufs[i])
            _scatter(c, bufs[i])
            _wait_scatter(bufs[(i + 3) % NBUF])
        _wait_scatter(bufs[(HCHUNK - 1) % NBUF])

    plsc.subcore_barrier()
    for p in range(RPT // RSTAGE):
        row0 = sid * RPT + p * RSTAGE
        pltpu.sync_copy(acc.at[pl.ds(row0, RSTAGE)], stage)
        pltpu.sync_copy(stage, out_hbm.at[cid, pl.ds(row0, RSTAGE)])


_sc_aggregate = pl.kernel(
    _sc_aggregate_body,
    out_type=jax.ShapeDtypeStruct((NC, ACC_N, HD), jnp.float32),
    mesh=plsc.VectorSubcoreMesh(core_axis_name="c", subcore_axis_name="s"),
    compiler_params=pltpu.CompilerParams(use_tc_tiling_on_sc=False),
    scratch_types=[
        pltpu.VMEM((HCHUNK, CH), jnp.int32),
        pltpu.VMEM((HCHUNK, CH), jnp.int32),
        pltpu.VMEM((HCHUNK, CH), jnp.float32),
        pltpu.VMEM((CH, HD), jnp.float32),
        pltpu.VMEM((CH, HD), jnp.float32),
        pltpu.VMEM((CH, HD), jnp.float32),
        pltpu.VMEM((CH, HD), jnp.float32),
        pltpu.VMEM((RSTAGE, HD), jnp.float32),
        pltpu.VMEM_SHARED((ACC_N, HD), jnp.float32),
        pltpu.SemaphoreType.DMA,
        pltpu.SemaphoreType.DMA,
    ],
)


BM = 2000


def _matmul_body(p_ref, k_ref, b_ref, o_ref):
    o_ref[...] = (
        jnp.dot(p_ref[0], k_ref[0:HD, :], preferred_element_type=jnp.float32)
        + jnp.dot(p_ref[1], k_ref[HD:D, :], preferred_element_type=jnp.float32)
        + b_ref[...]
    )


def _matmul(agg, k, bias2d):
    return pl.pallas_call(
        _matmul_body,
        out_shape=jax.ShapeDtypeStruct((N, D), jnp.float32),
        grid=(N // BM,),
        in_specs=[
            pl.BlockSpec((NC, BM, HD), lambda i: (0, i, 0)),
            pl.BlockSpec((D, D), lambda i: (0, 0)),
            pl.BlockSpec((1, D), lambda i: (0, 0)),
        ],
        out_specs=pl.BlockSpec((BM, D), lambda i: (i, 0)),
    )(agg, k, bias2d)


@jax.jit
def kernel(x, edge_index, edge_weight, kernel, bias):
    npad = EPAD - E
    src = jnp.concatenate(
        [edge_index[1].astype(jnp.int32), jnp.zeros((npad,), jnp.int32)]
    ).reshape(NS, NCHUNK, CH)
    dst = jnp.concatenate(
        [edge_index[0].astype(jnp.int32),
         jnp.full((npad,), ACC_N - 1, jnp.int32)]
    ).reshape(NS, NCHUNK, CH)
    w = jnp.concatenate(
        [edge_weight, jnp.zeros((npad,), jnp.float32)]
    ).reshape(NS, NCHUNK, CH)
    srcs = jnp.stack([src, src + N])
    xs = jnp.concatenate([x[:, :HD], x[:, HD:]], axis=0)
    agg = _sc_aggregate(xs, srcs, dst, w)
    return _matmul(agg, kernel, bias.reshape(1, D))


# speedup vs baseline: 1.2209x; 1.2209x over previous
import jax
import jax.numpy as jnp
from jax import lax
from jax.experimental import pallas as pl
from jax.experimental.pallas import tpu as pltpu
from jax.experimental.pallas import tpu_sc as plsc

N = 10000
E = 320000
D = 128
HD = D // 2

NC = 2
NS = 16
CH = 128
NCHUNK = 160
EPW = NCHUNK * CH
EPAD = NS * EPW
ACC_N = 10240
RPT = ACC_N // NS
RSTAGE = 128
NBUF = 4
HCHUNK = 40
XPT = N // NS


def _sc_aggregate_body(xs_hbm, srcs_hbm, dsts_hbm, ws_hbm, out_hbm,
                       src_v, dst_v, w_v, buf0, buf1, buf2, buf3,
                       acc, xsp, gsem, ssem):
    bufs = [buf0, buf1, buf2, buf3]
    stage = buf0
    cid = lax.axis_index("c")
    sid = lax.axis_index("s")

    pltpu.sync_copy(xs_hbm.at[pl.ds(cid * N + sid * XPT, XPT)],
                    xsp.at[pl.ds(sid * XPT, XPT)])

    zero16 = jnp.zeros((16,), jnp.float32)

    def _zero_row(i, _):
        for r in range(HD // 16):
            stage[i, pl.ds(r * 16, 16)] = zero16
        return 0

    lax.fori_loop(0, RSTAGE, _zero_row, 0)
    for p in range(RPT // RSTAGE):
        pltpu.sync_copy(stage, acc.at[pl.ds(sid * RPT + p * RSTAGE, RSTAGE)])
    plsc.subcore_barrier()

    def _gather(c, rows):
        pltpu.async_copy(xsp.at[src_v.at[c]], rows, gsem)

    def _wait_gather(rows):
        pltpu.make_async_copy(xsp.at[src_v.at[0]], rows, gsem).wait()

    def _scatter(c, rows):
        pltpu.async_copy(rows, acc.at[dst_v.at[c]], ssem, add=True)

    def _wait_scatter(rows):
        pltpu.make_async_copy(rows, acc.at[dst_v.at[0]], ssem).wait()

    def _scale(c, rows):
        def _edge_group(g, _):
            wv = w_v[c, pl.ds(g * 16, 16)]
            for l in range(16):
                j = g * 16 + l
                w = wv[l]
                for r in range(HD // 16):
                    rows[j, pl.ds(r * 16, 16)] = rows[j, pl.ds(r * 16, 16)] * w
            return 0

        lax.fori_loop(0, CH // 16, _edge_group, 0)

    for h in range(NCHUNK // HCHUNK):
        pltpu.sync_copy(srcs_hbm.at[0, sid, pl.ds(h * HCHUNK, HCHUNK)],
                        src_v)
        pltpu.sync_copy(dsts_hbm.at[sid, pl.ds(h * HCHUNK, HCHUNK)], dst_v)
        pltpu.sync_copy(ws_hbm.at[sid, pl.ds(h * HCHUNK, HCHUNK)], w_v)

        for b in range(NBUF - 1):
            _gather(b, bufs[b])

        _wait_gather(bufs[0])
        _scale(0, bufs[0])
        _scatter(0, bufs[0])
        _gather(NBUF - 1, bufs[NBUF - 1])

        def _step(c, i):
            _wait_gather(bufs[i])
            _scale(c, bufs[i])
            _scatter(c, bufs[i])
            _wait_scatter(bufs[(i + 3) % NBUF])
            _gather(c + NBUF - 1, bufs[(i + 3) % NBUF])

        def _quad(t, _):
            for i in range(NBUF):
                c = t * NBUF + 1 + i
                _step(c, (1 + i) % NBUF)
            return 0

        lax.fori_loop(0, (HCHUNK - NBUF) // NBUF, _quad, 0)

        for c in range(HCHUNK - 3, HCHUNK):
            i = c % NBUF
            _wait_gather(bufs[i])
            _scale(c, bufs[i])
            _scatter(c, bufs[i])
            _wait_scatter(bufs[(i + 3) % NBUF])
        _wait_scatter(bufs[(HCHUNK - 1) % NBUF])

    plsc.subcore_barrier()
    for p in range(RPT // RSTAGE):
        row0 = sid * RPT + p * RSTAGE
        pltpu.sync_copy(acc.at[pl.ds(row0, RSTAGE)], stage)
        pltpu.sync_copy(stage, out_hbm.at[cid, pl.ds(row0, RSTAGE)])


_sc_aggregate = pl.kernel(
    _sc_aggregate_body,
    out_type=jax.ShapeDtypeStruct((NC, ACC_N, HD), jnp.float32),
    mesh=plsc.VectorSubcoreMesh(core_axis_name="c", subcore_axis_name="s"),
    compiler_params=pltpu.CompilerParams(use_tc_tiling_on_sc=False),
    scratch_types=[
        pltpu.VMEM((HCHUNK, CH), jnp.int32),
        pltpu.VMEM((HCHUNK, CH), jnp.int32),
        pltpu.VMEM((HCHUNK, CH), jnp.float32),
        pltpu.VMEM((CH, HD), jnp.float32),
        pltpu.VMEM((CH, HD), jnp.float32),
        pltpu.VMEM((CH, HD), jnp.float32),
        pltpu.VMEM((CH, HD), jnp.float32),
        pltpu.VMEM_SHARED((ACC_N, HD), jnp.float32),
        pltpu.VMEM_SHARED((N, HD), jnp.float32),
        pltpu.SemaphoreType.DMA,
        pltpu.SemaphoreType.DMA,
    ],
)


BM = 2000


def _matmul_body(p_ref, k_ref, b_ref, o_ref):
    o_ref[...] = (
        jnp.dot(p_ref[0], k_ref[0:HD, :], preferred_element_type=jnp.float32)
        + jnp.dot(p_ref[1], k_ref[HD:D, :], preferred_element_type=jnp.float32)
        + b_ref[...]
    )


def _matmul(agg, k, bias2d):
    return pl.pallas_call(
        _matmul_body,
        out_shape=jax.ShapeDtypeStruct((N, D), jnp.float32),
        grid=(N // BM,),
        in_specs=[
            pl.BlockSpec((NC, BM, HD), lambda i: (0, i, 0)),
            pl.BlockSpec((D, D), lambda i: (0, 0)),
            pl.BlockSpec((1, D), lambda i: (0, 0)),
        ],
        out_specs=pl.BlockSpec((BM, D), lambda i: (i, 0)),
    )(agg, k, bias2d)


@jax.jit
def kernel(x, edge_index, edge_weight, kernel, bias):
    npad = EPAD - E
    src = jnp.concatenate(
        [edge_index[1].astype(jnp.int32), jnp.zeros((npad,), jnp.int32)]
    ).reshape(NS, NCHUNK, CH)
    dst = jnp.concatenate(
        [edge_index[0].astype(jnp.int32),
         jnp.full((npad,), ACC_N - 1, jnp.int32)]
    ).reshape(NS, NCHUNK, CH)
    w = jnp.concatenate(
        [edge_weight, jnp.zeros((npad,), jnp.float32)]
    ).reshape(NS, NCHUNK, CH)
    srcs = src.reshape(1, NS, NCHUNK, CH)
    xs = jnp.concatenate([x[:, :HD], x[:, HD:]], axis=0)
    agg = _sc_aggregate(xs, srcs, dst, w)
    return _matmul(agg, kernel, bias.reshape(1, D))


# speedup vs baseline: 2.2644x; 1.8546x over previous
import jax
import jax.numpy as jnp
from jax import lax
from jax.experimental import pallas as pl
from jax.experimental.pallas import tpu as pltpu
from jax.experimental.pallas import tpu_sc as plsc

N = 10000
E = 320000
D = 128
HD = D // 2

NC = 2
NS = 16
CH = 128
NCHUNK = 160
EPW = NCHUNK * CH
EPAD = NS * EPW
ACC_N = 10240
RPT = ACC_N // NS
RSTAGE = 128
NBUF = 4
HCHUNK = 40
XPT = N // NS


def _sc_aggregate_body(xs_hbm, srcs_hbm, dsts_hbm, ws_hbm, out_hbm,
                       src_v, dst_v, w_v, buf0, buf1, buf2, buf3,
                       acc, xsp, gsem, ssem):
    bufs = [buf0, buf1, buf2, buf3]
    stage = buf0
    cid = lax.axis_index("c")
    sid = lax.axis_index("s")

    pltpu.sync_copy(xs_hbm.at[pl.ds(cid * N + sid * XPT, XPT)],
                    xsp.at[pl.ds(sid * XPT, XPT)])

    zero16 = jnp.zeros((16,), jnp.float32)

    def _zero_row(i, _):
        for r in range(HD // 16):
            stage[i, pl.ds(r * 16, 16)] = zero16
        return 0

    lax.fori_loop(0, RSTAGE, _zero_row, 0)
    for p in range(RPT // RSTAGE):
        pltpu.sync_copy(stage, acc.at[pl.ds(sid * RPT + p * RSTAGE, RSTAGE)])
    plsc.subcore_barrier()

    def _gather(c, rows):
        pltpu.async_copy(xsp.at[src_v.at[c]], rows, gsem)

    def _wait_gather(rows):
        pltpu.make_async_copy(xsp.at[src_v.at[0]], rows, gsem).wait()

    def _scatter(c, rows):
        pltpu.async_copy(rows, acc.at[dst_v.at[c]], ssem, add=True)

    def _wait_scatter(rows):
        pltpu.make_async_copy(rows, acc.at[dst_v.at[0]], ssem).wait()

    def _scale(c, rows):
        @plsc.parallel_loop(0, CH // 16, unroll=2)
        def _edge_group(g):
            wv = w_v[c, pl.ds(g * 16, 16)]
            for l in range(16):
                j = g * 16 + l
                w = wv[l]
                for r in range(HD // 16):
                    rows[j, pl.ds(r * 16, 16)] = rows[j, pl.ds(r * 16, 16)] * w

    for h in range(NCHUNK // HCHUNK):
        pltpu.sync_copy(srcs_hbm.at[0, sid, pl.ds(h * HCHUNK, HCHUNK)],
                        src_v)
        pltpu.sync_copy(dsts_hbm.at[sid, pl.ds(h * HCHUNK, HCHUNK)], dst_v)
        pltpu.sync_copy(ws_hbm.at[sid, pl.ds(h * HCHUNK, HCHUNK)], w_v)

        for b in range(NBUF - 1):
            _gather(b, bufs[b])

        _wait_gather(bufs[0])
        _scale(0, bufs[0])
        _scatter(0, bufs[0])
        _gather(NBUF - 1, bufs[NBUF - 1])

        def _step(c, i):
            _wait_gather(bufs[i])
            _scale(c, bufs[i])
            _scatter(c, bufs[i])
            _wait_scatter(bufs[(i + 3) % NBUF])
            _gather(c + NBUF - 1, bufs[(i + 3) % NBUF])

        def _quad(t, _):
            for i in range(NBUF):
                c = t * NBUF + 1 + i
                _step(c, (1 + i) % NBUF)
            return 0

        lax.fori_loop(0, (HCHUNK - NBUF) // NBUF, _quad, 0)

        for c in range(HCHUNK - 3, HCHUNK):
            i = c % NBUF
            _wait_gather(bufs[i])
            _scale(c, bufs[i])
            _scatter(c, bufs[i])
            _wait_scatter(bufs[(i + 3) % NBUF])
        _wait_scatter(bufs[(HCHUNK - 1) % NBUF])

    plsc.subcore_barrier()
    for p in range(RPT // RSTAGE):
        row0 = sid * RPT + p * RSTAGE
        pltpu.sync_copy(acc.at[pl.ds(row0, RSTAGE)], stage)
        pltpu.sync_copy(stage, out_hbm.at[cid, pl.ds(row0, RSTAGE)])


_sc_aggregate = pl.kernel(
    _sc_aggregate_body,
    out_type=jax.ShapeDtypeStruct((NC, ACC_N, HD), jnp.float32),
    mesh=plsc.VectorSubcoreMesh(core_axis_name="c", subcore_axis_name="s"),
    compiler_params=pltpu.CompilerParams(use_tc_tiling_on_sc=False),
    scratch_types=[
        pltpu.VMEM((HCHUNK, CH), jnp.int32),
        pltpu.VMEM((HCHUNK, CH), jnp.int32),
        pltpu.VMEM((HCHUNK, CH), jnp.float32),
        pltpu.VMEM((CH, HD), jnp.float32),
        pltpu.VMEM((CH, HD), jnp.float32),
        pltpu.VMEM((CH, HD), jnp.float32),
        pltpu.VMEM((CH, HD), jnp.float32),
        pltpu.VMEM_SHARED((ACC_N, HD), jnp.float32),
        pltpu.VMEM_SHARED((N, HD), jnp.float32),
        pltpu.SemaphoreType.DMA,
        pltpu.SemaphoreType.DMA,
    ],
)


BM = 2000


def _matmul_body(p_ref, k_ref, b_ref, o_ref):
    o_ref[...] = (
        jnp.dot(p_ref[0], k_ref[0:HD, :], preferred_element_type=jnp.float32)
        + jnp.dot(p_ref[1], k_ref[HD:D, :], preferred_element_type=jnp.float32)
        + b_ref[...]
    )


def _matmul(agg, k, bias2d):
    return pl.pallas_call(
        _matmul_body,
        out_shape=jax.ShapeDtypeStruct((N, D), jnp.float32),
        grid=(N // BM,),
        in_specs=[
            pl.BlockSpec((NC, BM, HD), lambda i: (0, i, 0)),
            pl.BlockSpec((D, D), lambda i: (0, 0)),
            pl.BlockSpec((1, D), lambda i: (0, 0)),
        ],
        out_specs=pl.BlockSpec((BM, D), lambda i: (i, 0)),
    )(agg, k, bias2d)


@jax.jit
def kernel(x, edge_index, edge_weight, kernel, bias):
    npad = EPAD - E
    src = jnp.concatenate(
        [edge_index[1].astype(jnp.int32), jnp.zeros((npad,), jnp.int32)]
    ).reshape(NS, NCHUNK, CH)
    dst = jnp.concatenate(
        [edge_index[0].astype(jnp.int32),
         jnp.full((npad,), ACC_N - 1, jnp.int32)]
    ).reshape(NS, NCHUNK, CH)
    w = jnp.concatenate(
        [edge_weight, jnp.zeros((npad,), jnp.float32)]
    ).reshape(NS, NCHUNK, CH)
    srcs = src.reshape(1, NS, NCHUNK, CH)
    xs = jnp.concatenate([x[:, :HD], x[:, HD:]], axis=0)
    agg = _sc_aggregate(xs, srcs, dst, w)
    return _matmul(agg, kernel, bias.reshape(1, D))


# speedup vs baseline: 2.4734x; 1.0923x over previous
import jax
import jax.numpy as jnp
from jax import lax
from jax.experimental import pallas as pl
from jax.experimental.pallas import tpu as pltpu
from jax.experimental.pallas import tpu_sc as plsc

N = 10000
E = 320000
D = 128
HD = D // 2

NC = 2
NS = 16
CH = 128
NCHUNK = 160
EPW = NCHUNK * CH
EPAD = NS * EPW
ACC_N = 10240
RPT = ACC_N // NS
RSTAGE = 128
NBUF = 4
HCHUNK = 40
XPT = N // NS


def _sc_aggregate_body(xs_hbm, srcs_hbm, dsts_hbm, ws_hbm, out_hbm,
                       src_v, dst_v, w_v, buf0, buf1, buf2, buf3,
                       acc, xsp, gsem, ssem):
    bufs = [buf0, buf1, buf2, buf3]
    stage = buf0
    cid = lax.axis_index("c")
    sid = lax.axis_index("s")

    for p in range(5):
        row0 = sid * XPT + p * 125
        pltpu.sync_copy(xs_hbm.at[pl.ds(row0, 125), pl.ds(cid * HD, HD)],
                        stage.at[pl.ds(0, 125)])
        pltpu.sync_copy(stage.at[pl.ds(0, 125)], xsp.at[pl.ds(row0, 125)])

    zero16 = jnp.zeros((16,), jnp.float32)

    def _zero_row(i, _):
        for r in range(HD // 16):
            stage[i, pl.ds(r * 16, 16)] = zero16
        return 0

    lax.fori_loop(0, RSTAGE, _zero_row, 0)
    for p in range(RPT // RSTAGE):
        pltpu.sync_copy(stage, acc.at[pl.ds(sid * RPT + p * RSTAGE, RSTAGE)])
    plsc.subcore_barrier()

    def _gather(c, rows):
        pltpu.async_copy(xsp.at[src_v.at[c]], rows, gsem)

    def _wait_gather(rows):
        pltpu.make_async_copy(xsp.at[src_v.at[0]], rows, gsem).wait()

    def _scatter(c, rows):
        pltpu.async_copy(rows, acc.at[dst_v.at[c]], ssem, add=True)

    def _wait_scatter(rows):
        pltpu.make_async_copy(rows, acc.at[dst_v.at[0]], ssem).wait()

    def _scale(c, rows):
        @plsc.parallel_loop(0, CH // 16, unroll=2)
        def _edge_group(g):
            wv = w_v[c, pl.ds(g * 16, 16)]
            for l in range(16):
                j = g * 16 + l
                w = wv[l]
                for r in range(HD // 16):
                    rows[j, pl.ds(r * 16, 16)] = rows[j, pl.ds(r * 16, 16)] * w

    for h in range(NCHUNK // HCHUNK):
        pltpu.sync_copy(srcs_hbm.at[0, sid, pl.ds(h * HCHUNK, HCHUNK)],
                        src_v)
        pltpu.sync_copy(dsts_hbm.at[sid, pl.ds(h * HCHUNK, HCHUNK)], dst_v)
        pltpu.sync_copy(ws_hbm.at[sid, pl.ds(h * HCHUNK, HCHUNK)], w_v)

        for b in range(NBUF - 1):
            _gather(b, bufs[b])

        _wait_gather(bufs[0])
        _scale(0, bufs[0])
        _scatter(0, bufs[0])
        _gather(NBUF - 1, bufs[NBUF - 1])

        def _step(c, i):
            _wait_gather(bufs[i])
            _scale(c, bufs[i])
            _scatter(c, bufs[i])
            _wait_scatter(bufs[(i + 3) % NBUF])
            _gather(c + NBUF - 1, bufs[(i + 3) % NBUF])

        def _quad(t, _):
            for i in range(NBUF):
                c = t * NBUF + 1 + i
                _step(c, (1 + i) % NBUF)
            return 0

        lax.fori_loop(0, (HCHUNK - NBUF) // NBUF, _quad, 0)

        for c in range(HCHUNK - 3, HCHUNK):
            i = c % NBUF
            _wait_gather(bufs[i])
            _scale(c, bufs[i])
            _scatter(c, bufs[i])
            _wait_scatter(bufs[(i + 3) % NBUF])
        _wait_scatter(bufs[(HCHUNK - 1) % NBUF])

    plsc.subcore_barrier()
    for p in range(RPT // RSTAGE):
        row0 = sid * RPT + p * RSTAGE
        pltpu.sync_copy(acc.at[pl.ds(row0, RSTAGE)], stage)
        pltpu.sync_copy(stage, out_hbm.at[cid, pl.ds(row0, RSTAGE)])


_sc_aggregate = pl.kernel(
    _sc_aggregate_body,
    out_type=jax.ShapeDtypeStruct((NC, ACC_N, HD), jnp.float32),
    mesh=plsc.VectorSubcoreMesh(core_axis_name="c", subcore_axis_name="s"),
    compiler_params=pltpu.CompilerParams(use_tc_tiling_on_sc=False),
    scratch_types=[
        pltpu.VMEM((HCHUNK, CH), jnp.int32),
        pltpu.VMEM((HCHUNK, CH), jnp.int32),
        pltpu.VMEM((HCHUNK, CH), jnp.float32),
        pltpu.VMEM((CH, HD), jnp.float32),
        pltpu.VMEM((CH, HD), jnp.float32),
        pltpu.VMEM((CH, HD), jnp.float32),
        pltpu.VMEM((CH, HD), jnp.float32),
        pltpu.VMEM_SHARED((ACC_N, HD), jnp.float32),
        pltpu.VMEM_SHARED((N, HD), jnp.float32),
        pltpu.SemaphoreType.DMA,
        pltpu.SemaphoreType.DMA,
    ],
)


BM = 2000


def _matmul_body(p_ref, k_ref, b_ref, o_ref):
    o_ref[...] = (
        jnp.dot(p_ref[0], k_ref[0:HD, :], preferred_element_type=jnp.float32)
        + jnp.dot(p_ref[1], k_ref[HD:D, :], preferred_element_type=jnp.float32)
        + b_ref[...]
    )


def _matmul(agg, k, bias2d):
    return pl.pallas_call(
        _matmul_body,
        out_shape=jax.ShapeDtypeStruct((N, D), jnp.float32),
        grid=(N // BM,),
        in_specs=[
            pl.BlockSpec((NC, BM, HD), lambda i: (0, i, 0)),
            pl.BlockSpec((D, D), lambda i: (0, 0)),
            pl.BlockSpec((1, D), lambda i: (0, 0)),
        ],
        out_specs=pl.BlockSpec((BM, D), lambda i: (i, 0)),
    )(agg, k, bias2d)


@jax.jit
def kernel(x, edge_index, edge_weight, kernel, bias):
    npad = EPAD - E
    src = jnp.concatenate(
        [edge_index[1].astype(jnp.int32), jnp.zeros((npad,), jnp.int32)]
    ).reshape(NS, NCHUNK, CH)
    dst = jnp.concatenate(
        [edge_index[0].astype(jnp.int32),
         N + jnp.arange(npad, dtype=jnp.int32) % (ACC_N - N)]
    ).reshape(NS, NCHUNK, CH)
    w = jnp.concatenate(
        [edge_weight, jnp.zeros((npad,), jnp.float32)]
    ).reshape(NS, NCHUNK, CH)
    srcs = src.reshape(1, NS, NCHUNK, CH)
    agg = _sc_aggregate(x, srcs, dst, w)
    return _matmul(agg, kernel, bias.reshape(1, D))
